# Initial kernel scaffold; baseline (speedup 1.0000x reference)
import jax
import jax.numpy as jnp
from jax.experimental import pallas as pl


def kernel(l, ab, ori, comp, y, idx, memory_l, memory_ab, memory_ori, memory_comp):
    raise NotImplementedError("write your pallas kernel here")


# speedup vs baseline: 1.3840x; 1.3840x over previous
import functools

import jax
import jax.numpy as jnp
from jax import lax
from jax.experimental import pallas as pl
from jax.experimental.pallas import tpu as pltpu
from jax.experimental.pallas import tpu_sc as plsc

_B = 1024
_D = 128
_OUT = 100000
_K1 = 513
_T = 0.07
_MOM = 0.5

_NC, _NS, _L = 2, 16, 16
_NW = _NC * _NS
_BPW = _B // _NW
_KP = 528
_NCH = _KP // _L
_NDB = _D // _L


def _sc_body(featcat_hbm, idxp_hbm, ml_hbm, mab_hbm, mo_hbm, mc_hbm,
             out_hbm,
             feat_v, idx_v, w_l, w_ab, w_o, w_c, outstage, sem):
    wid = lax.axis_index("s") * _NC + lax.axis_index("c")
    iota = lax.broadcasted_iota(jnp.int32, (_L,), 0)

    def b_loop(bi, carry):
        b = wid * _BPW + bi
        pltpu.sync_copy(idxp_hbm.at[b], idx_v)
        pltpu.sync_copy(featcat_hbm.at[b], feat_v)

        def c_loop(c, carry2):
            isl = idx_v.at[pl.ds(c * _L, _L)]
            d1 = pltpu.async_copy(ml_hbm.at[isl], w_l, sem)
            d2 = pltpu.async_copy(mab_hbm.at[isl], w_ab, sem)
            d3 = pltpu.async_copy(mo_hbm.at[isl], w_o, sem)
            d4 = pltpu.async_copy(mc_hbm.at[isl], w_c, sem)
            d1.wait()
            d2.wait()
            d3.wait()
            d4.wait()

            def d_loop(dd, accs):
                a0, a1, a2, a3, a4, a5 = accs
                d0 = dd * _L
                vfl = feat_v[pl.ds(d0, _L)]
                vfab = feat_v[pl.ds(_D + d0, _L)]
                vfo = feat_v[pl.ds(2 * _D + d0, _L)]
                vfc = feat_v[pl.ds(3 * _D + d0, _L)]
                for u in range(_L):
                    dvec = jnp.full((_L,), 1, jnp.int32) * (d0 + u)
                    cl = plsc.load_gather(w_l, [iota, dvec])
                    cab = plsc.load_gather(w_ab, [iota, dvec])
                    co = plsc.load_gather(w_o, [iota, dvec])
                    cc = plsc.load_gather(w_c, [iota, dvec])
                    fl = vfl[u]
                    fab = vfab[u]
                    fo = vfo[u]
                    fc = vfc[u]
                    a0 = a0 + co * fl
                    a1 = a1 + cl * fab
                    a2 = a2 + cab * fo
                    a3 = a3 + cc * fab
                    a4 = a4 + cc * fl
                    a5 = a5 + co * fc
                return (a0, a1, a2, a3, a4, a5)

            z = jnp.zeros((_L,), jnp.float32)
            accs = lax.fori_loop(0, _NDB, d_loop, (z, z, z, z, z, z))
            for t in range(6):
                outstage[t, pl.ds(c * _L, _L)] = accs[t] / _T
            return carry2

        lax.fori_loop(0, _NCH, c_loop, 0)
        for t in range(6):
            pltpu.sync_copy(outstage.at[t], out_hbm.at[t, b])
        return carry

    lax.fori_loop(0, _BPW, b_loop, 0)


def _copy_body(a, b, c, d, oa, ob, oc, od):
    oa[...] = a[...]
    ob[...] = b[...]
    oc[...] = c[...]
    od[...] = d[...]


def _upd_body(y_ref, ml, mab, mo, mc, fl, fab, fo, fc,
              cl, cab, co, cc, ol, oab, oo, oc):
    del y_ref, cl, cab, co, cc
    for m, f, o in ((ml, fl, ol), (mab, fab, oab), (mo, fo, oo), (mc, fc, oc)):
        pos = m[...] * _MOM + f[...] * (1.0 - _MOM)
        norm = jnp.sqrt(jnp.sum(pos * pos, axis=-1, keepdims=True))
        o[...] = pos / norm


def kernel(l, ab, ori, comp, y, idx, memory_l, memory_ab, memory_ori, memory_comp):
    y = y.astype(jnp.int32)
    idxp = jnp.concatenate(
        [y[:, None], idx[:, 1:].astype(jnp.int32),
         jnp.zeros((_B, _KP - _K1), jnp.int32)], axis=1)
    featcat = jnp.concatenate([l, ab, ori, comp], axis=1)

    sc_fn = pl.kernel(
        _sc_body,
        out_type=jax.ShapeDtypeStruct((6, _B, _KP), jnp.float32),
        mesh=plsc.VectorSubcoreMesh(core_axis_name="c", subcore_axis_name="s"),
        compiler_params=pltpu.CompilerParams(needs_layout_passes=False),
        scratch_types=[
            pltpu.VMEM((4 * _D,), jnp.float32),
            pltpu.VMEM((_KP,), jnp.int32),
            pltpu.VMEM((_L, _D), jnp.float32),
            pltpu.VMEM((_L, _D), jnp.float32),
            pltpu.VMEM((_L, _D), jnp.float32),
            pltpu.VMEM((_L, _D), jnp.float32),
            pltpu.VMEM((6, _KP), jnp.float32),
            pltpu.SemaphoreType.DMA,
        ],
    )
    outs = sc_fn(featcat, idxp, memory_l, memory_ab, memory_ori, memory_comp)
    outs = outs[:, :, :_K1]

    rows_blk = 1000
    bank_sds = jax.ShapeDtypeStruct((_OUT, _D), jnp.float32)
    blk = pl.BlockSpec((rows_blk, _D), lambda i: (i, 0))
    copies = pl.pallas_call(
        _copy_body,
        grid=(_OUT // rows_blk,),
        in_specs=[blk] * 4,
        out_specs=[blk] * 4,
        out_shape=[bank_sds] * 4,
    )(memory_l, memory_ab, memory_ori, memory_comp)

    banks3 = [m.reshape(_OUT, 1, _D)
              for m in (memory_l, memory_ab, memory_ori, memory_comp)]
    feats3 = [f.reshape(_B, 1, _D) for f in (l, ab, ori, comp)]
    copies3 = [c.reshape(_OUT, 1, _D) for c in copies]
    bank3_sds = jax.ShapeDtypeStruct((_OUT, 1, _D), jnp.float32)
    row_y = pl.BlockSpec((1, 1, _D), lambda i, yref: (yref[i], 0, 0))
    row_i = pl.BlockSpec((1, 1, _D), lambda i, yref: (i, 0, 0))
    anyspec = pl.BlockSpec(memory_space=pl.ANY)
    grid_spec = pltpu.PrefetchScalarGridSpec(
        num_scalar_prefetch=1,
        grid=(_B,),
        in_specs=[row_y] * 4 + [row_i] * 4 + [anyspec] * 4,
        out_specs=[row_y] * 4,
    )
    new_banks = pl.pallas_call(
        _upd_body,
        grid_spec=grid_spec,
        out_shape=[bank3_sds] * 4,
        input_output_aliases={9: 0, 10: 1, 11: 2, 12: 3},
    )(y, *banks3, *feats3, *copies3)
    new_banks = [nb.reshape(_OUT, _D) for nb in new_banks]

    return (outs[0][..., None], outs[1][..., None], outs[2][..., None],
            outs[3][..., None], outs[4][..., None], outs[5][..., None],
            new_banks[0], new_banks[1], new_banks[2], new_banks[3])


# speedup vs baseline: 1.6239x; 1.1733x over previous
import jax
import jax.numpy as jnp
from jax import lax
from jax.experimental import pallas as pl
from jax.experimental.pallas import tpu as pltpu
from jax.experimental.pallas import tpu_sc as plsc

_B = 1024
_D = 128
_OUT = 100000
_K1 = 513
_T = 0.07
_MOM = 0.5

_NC, _NS, _L = 2, 16, 16
_NW = _NC * _NS
_BPW = _B // _NW
_S = 48
_KP = 528
_NCH = _KP // _S
_NG = _S // _L
_NDB = _D // _L


def _sc_body(featcat_hbm, idxp_hbm, ml_hbm, mab_hbm, mo_hbm, mc_hbm,
             out_hbm,
             idx_all, feat_all,
             wa_l, wa_ab, wa_o, wa_c, wb_l, wb_ab, wb_o, wb_c,
             outstage, sem_a, sem_b, sem_o):
    wid = lax.axis_index("s") * _NC + lax.axis_index("c")
    iota = lax.broadcasted_iota(jnp.int32, (_L,), 0)
    banks = (ml_hbm, mab_hbm, mo_hbm, mc_hbm)
    buf_a = (wa_l, wa_ab, wa_o, wa_c)
    buf_b = (wb_l, wb_ab, wb_o, wb_c)

    pltpu.sync_copy(idxp_hbm.at[pl.ds(wid * _BPW * _KP, _BPW * _KP)], idx_all)
    pltpu.sync_copy(featcat_hbm.at[pl.ds(wid * _BPW * 4 * _D, _BPW * 4 * _D)],
                    feat_all)

    def issue(bi, c, bufs, sem):
        isl = idx_all.at[pl.ds(bi * _KP + c * _S, _S)]
        for m, w in zip(banks, bufs):
            pltpu.async_copy(m.at[isl], w, sem)

    def drain(bi, c, bufs, sem):
        isl = idx_all.at[pl.ds(bi * _KP + c * _S, _S)]
        for m, w in zip(banks, bufs):
            pltpu.make_async_copy(m.at[isl], w, sem).wait()

    def compute(bi, c, bufs):
        wl, wab, wo, wc = bufs
        slot = lax.rem(bi, 2)

        def g_loop(g, _):
            riota = iota + g * _L

            def d_loop(dd, accs):
                a0, a1, a2, a3, a4, a5 = accs
                d0 = dd * _L
                fb = bi * 4 * _D
                vfl = feat_all[pl.ds(fb + d0, _L)]
                vfab = feat_all[pl.ds(fb + _D + d0, _L)]
                vfo = feat_all[pl.ds(fb + 2 * _D + d0, _L)]
                vfc = feat_all[pl.ds(fb + 3 * _D + d0, _L)]
                one = jnp.full((_L,), 1, jnp.int32)
                for u in range(_L):
                    dvec = one * (d0 + u)
                    cl = plsc.load_gather(wl, [riota, dvec])
                    cab = plsc.load_gather(wab, [riota, dvec])
                    co = plsc.load_gather(wo, [riota, dvec])
                    cc = plsc.load_gather(wc, [riota, dvec])
                    fl = vfl[u]
                    fab = vfab[u]
                    fo = vfo[u]
                    fc = vfc[u]
                    a0 = a0 + co * fl
                    a1 = a1 + cl * fab
                    a2 = a2 + cab * fo
                    a3 = a3 + cc * fab
                    a4 = a4 + cc * fl
                    a5 = a5 + co * fc
                return (a0, a1, a2, a3, a4, a5)

            z = jnp.zeros((_L,), jnp.float32)
            accs = lax.fori_loop(0, _NDB, d_loop, (z, z, z, z, z, z))
            for t in range(6):
                outstage[pl.ds(slot * 6 * _KP + t * _KP + c * _S + g * _L,
                               _L)] = accs[t] / _T
            return _

        lax.fori_loop(0, _NG, g_loop, 0)

    def b_loop(bi, carry):
        gb = wid * _BPW + bi
        slot = lax.rem(bi, 2)

        @pl.when(bi >= 2)
        def _wait_prev():
            pltpu.make_async_copy(outstage.at[pl.ds(slot * 6 * _KP, 6 * _KP)],
                                  out_hbm.at[pl.ds((gb - 2) * 6 * _KP, 6 * _KP)], sem_o).wait()

        issue(bi, 0, buf_a, sem_a)

        def t_loop(t, _):
            c0 = 2 * t
            issue(bi, c0 + 1, buf_b, sem_b)
            drain(bi, c0, buf_a, sem_a)
            compute(bi, c0, buf_a)
            issue(bi, c0 + 2, buf_a, sem_a)
            drain(bi, c0 + 1, buf_b, sem_b)
            compute(bi, c0 + 1, buf_b)
            return _

        lax.fori_loop(0, (_NCH - 1) // 2, t_loop, 0)
        drain(bi, _NCH - 1, buf_a, sem_a)
        compute(bi, _NCH - 1, buf_a)

        pltpu.async_copy(outstage.at[pl.ds(slot * 6 * _KP, 6 * _KP)],
                         out_hbm.at[pl.ds(gb * 6 * _KP, 6 * _KP)], sem_o)
        return carry

    lax.fori_loop(0, _BPW, b_loop, 0)
    pltpu.make_async_copy(outstage.at[pl.ds(0, 6 * _KP)],
                          out_hbm.at[pl.ds((wid * _BPW + _BPW - 2) * 6 * _KP, 6 * _KP)], sem_o).wait()
    pltpu.make_async_copy(outstage.at[pl.ds(6 * _KP, 6 * _KP)],
                          out_hbm.at[pl.ds((wid * _BPW + _BPW - 1) * 6 * _KP, 6 * _KP)], sem_o).wait()


def _copy_body(a, b, c, d, oa, ob, oc, od):
    oa[...] = a[...]
    ob[...] = b[...]
    oc[...] = c[...]
    od[...] = d[...]


def _upd_body(y_ref, ml, mab, mo, mc, fl, fab, fo, fc,
              cl, cab, co, cc, ol, oab, oo, oc):
    del y_ref, cl, cab, co, cc
    for m, f, o in ((ml, fl, ol), (mab, fab, oab), (mo, fo, oo), (mc, fc, oc)):
        pos = m[...] * _MOM + f[...] * (1.0 - _MOM)
        norm = jnp.sqrt(jnp.sum(pos * pos, axis=-1, keepdims=True))
        o[...] = pos / norm


def kernel(l, ab, ori, comp, y, idx, memory_l, memory_ab, memory_ori, memory_comp):
    y = y.astype(jnp.int32)
    idxp = jnp.concatenate(
        [y[:, None], idx[:, 1:].astype(jnp.int32),
         jnp.zeros((_B, _KP - _K1), jnp.int32)], axis=1)
    featcat = jnp.concatenate([l, ab, ori, comp], axis=1)

    sc_fn = pl.kernel(
        _sc_body,
        out_type=jax.ShapeDtypeStruct((_B * 6 * _KP,), jnp.float32),
        mesh=plsc.VectorSubcoreMesh(core_axis_name="c", subcore_axis_name="s"),
        compiler_params=pltpu.CompilerParams(needs_layout_passes=False),
        scratch_types=[
            pltpu.VMEM((_BPW * _KP,), jnp.int32),
            pltpu.VMEM((_BPW * 4 * _D,), jnp.float32),
            pltpu.VMEM((_S, _D), jnp.float32),
            pltpu.VMEM((_S, _D), jnp.float32),
            pltpu.VMEM((_S, _D), jnp.float32),
            pltpu.VMEM((_S, _D), jnp.float32),
            pltpu.VMEM((_S, _D), jnp.float32),
            pltpu.VMEM((_S, _D), jnp.float32),
            pltpu.VMEM((_S, _D), jnp.float32),
            pltpu.VMEM((_S, _D), jnp.float32),
            pltpu.VMEM((2 * 6 * _KP,), jnp.float32),
            pltpu.SemaphoreType.DMA,
            pltpu.SemaphoreType.DMA,
            pltpu.SemaphoreType.DMA,
        ],
    )
    sc_out = sc_fn(featcat.reshape(-1), idxp.reshape(-1),
                   memory_l, memory_ab, memory_ori, memory_comp)

    rows_blk = 1000
    bank_sds = jax.ShapeDtypeStruct((_OUT, _D), jnp.float32)
    blk = pl.BlockSpec((rows_blk, _D), lambda i: (i, 0))
    copies = pl.pallas_call(
        _copy_body,
        grid=(_OUT // rows_blk,),
        in_specs=[blk] * 4,
        out_specs=[blk] * 4,
        out_shape=[bank_sds] * 4,
    )(memory_l, memory_ab, memory_ori, memory_comp)

    banks3 = [m.reshape(_OUT, 1, _D)
              for m in (memory_l, memory_ab, memory_ori, memory_comp)]
    feats3 = [f.reshape(_B, 1, _D) for f in (l, ab, ori, comp)]
    copies3 = [c.reshape(_OUT, 1, _D) for c in copies]
    bank3_sds = jax.ShapeDtypeStruct((_OUT, 1, _D), jnp.float32)
    row_y = pl.BlockSpec((1, 1, _D), lambda i, yref: (yref[i], 0, 0))
    row_i = pl.BlockSpec((1, 1, _D), lambda i, yref: (i, 0, 0))
    anyspec = pl.BlockSpec(memory_space=pl.ANY)
    grid_spec = pltpu.PrefetchScalarGridSpec(
        num_scalar_prefetch=1,
        grid=(_B,),
        in_specs=[row_y] * 4 + [row_i] * 4 + [anyspec] * 4,
        out_specs=[row_y] * 4,
    )
    new_banks = pl.pallas_call(
        _upd_body,
        grid_spec=grid_spec,
        out_shape=[bank3_sds] * 4,
        input_output_aliases={9: 0, 10: 1, 11: 2, 12: 3},
    )(y, *banks3, *feats3, *copies3)
    new_banks = [nb.reshape(_OUT, _D) for nb in new_banks]

    outs = sc_out.reshape(_B, 6, _KP)[:, :, :_K1]
    return (outs[:, 0][..., None], outs[:, 1][..., None], outs[:, 2][..., None],
            outs[:, 3][..., None], outs[:, 4][..., None], outs[:, 5][..., None],
            new_banks[0], new_banks[1], new_banks[2], new_banks[3])


# speedup vs baseline: 1.6290x; 1.0031x over previous
import jax
import jax.numpy as jnp
from jax import lax
from jax.experimental import pallas as pl
from jax.experimental.pallas import tpu as pltpu
from jax.experimental.pallas import tpu_sc as plsc

_B = 1024
_D = 128
_OUT = 100000
_K1 = 513
_T = 0.07
_MOM = 0.5

_NC, _NS, _L = 2, 16, 16
_NW = _NC * _NS
_BPW = _B // _NW
_S = 48
_KP = 528
_NCH = _KP // _S
_NG = _S // _L
_NDB = _D // _L


def _sc_body(featcat_hbm, idxp_hbm, ml_hbm, mab_hbm, mo_hbm, mc_hbm,
             out_hbm,
             idx_all, feat_all,
             wa_l, wa_ab, wa_o, wa_c, wb_l, wb_ab, wb_o, wb_c,
             outstage, sa0, sa1, sa2, sa3, sb0, sb1, sb2, sb3, sem_o):
    wid = lax.axis_index("s") * _NC + lax.axis_index("c")
    iota = lax.broadcasted_iota(jnp.int32, (_L,), 0)
    banks = (ml_hbm, mab_hbm, mo_hbm, mc_hbm)
    buf_a = (wa_l, wa_ab, wa_o, wa_c)
    buf_b = (wb_l, wb_ab, wb_o, wb_c)
    sems_a = (sa0, sa1, sa2, sa3)
    sems_b = (sb0, sb1, sb2, sb3)

    pltpu.sync_copy(idxp_hbm.at[pl.ds(wid * _BPW * _KP, _BPW * _KP)], idx_all)
    pltpu.sync_copy(featcat_hbm.at[pl.ds(wid * _BPW * 4 * _D, _BPW * 4 * _D)],
                    feat_all)

    def issue(bi, c, bufs, sems):
        isl = idx_all.at[pl.ds(bi * _KP + c * _S, _S)]
        for m, w, sem in zip(banks, bufs, sems):
            pltpu.async_copy(m.at[isl], w, sem)

    def drain(bi, c, bufs, sems):
        isl = idx_all.at[pl.ds(bi * _KP + c * _S, _S)]
        for m, w, sem in zip(banks, bufs, sems):
            pltpu.make_async_copy(m.at[isl], w, sem).wait()

    def compute(bi, c, bufs):
        wl, wab, wo, wc = bufs
        slot = lax.rem(bi, 2)

        def g_loop(g, _):
            riota = iota + g * _L

            def d_loop(dd, accs):
                a0, a1, a2, a3, a4, a5 = accs
                d0 = dd * _L
                fb = bi * 4 * _D
                vfl = feat_all[pl.ds(fb + d0, _L)]
                vfab = feat_all[pl.ds(fb + _D + d0, _L)]
                vfo = feat_all[pl.ds(fb + 2 * _D + d0, _L)]
                vfc = feat_all[pl.ds(fb + 3 * _D + d0, _L)]
                one = jnp.full((_L,), 1, jnp.int32)
                for u in range(_L):
                    dvec = one * (d0 + u)
                    cl = plsc.load_gather(wl, [riota, dvec])
                    cab = plsc.load_gather(wab, [riota, dvec])
                    co = plsc.load_gather(wo, [riota, dvec])
                    cc = plsc.load_gather(wc, [riota, dvec])
                    fl = vfl[u]
                    fab = vfab[u]
                    fo = vfo[u]
                    fc = vfc[u]
                    a0 = a0 + co * fl
                    a1 = a1 + cl * fab
                    a2 = a2 + cab * fo
                    a3 = a3 + cc * fab
                    a4 = a4 + cc * fl
                    a5 = a5 + co * fc
                return (a0, a1, a2, a3, a4, a5)

            z = jnp.zeros((_L,), jnp.float32)
            accs = lax.fori_loop(0, _NDB, d_loop, (z, z, z, z, z, z))
            for t in range(6):
                outstage[pl.ds(slot * 6 * _KP + t * _KP + c * _S + g * _L,
                               _L)] = accs[t] / _T
            return _

        lax.fori_loop(0, _NG, g_loop, 0)

    def b_loop(bi, carry):
        gb = wid * _BPW + bi
        slot = lax.rem(bi, 2)

        @pl.when(bi >= 2)
        def _wait_prev():
            pltpu.make_async_copy(outstage.at[pl.ds(slot * 6 * _KP, 6 * _KP)],
                                  out_hbm.at[pl.ds((gb - 2) * 6 * _KP, 6 * _KP)], sem_o).wait()

        issue(bi, 0, buf_a, sems_a)

        def t_loop(t, _):
            c0 = 2 * t
            issue(bi, c0 + 1, buf_b, sems_b)
            drain(bi, c0, buf_a, sems_a)
            compute(bi, c0, buf_a)
            issue(bi, c0 + 2, buf_a, sems_a)
            drain(bi, c0 + 1, buf_b, sems_b)
            compute(bi, c0 + 1, buf_b)
            return _

        lax.fori_loop(0, (_NCH - 1) // 2, t_loop, 0)
        drain(bi, _NCH - 1, buf_a, sems_a)
        compute(bi, _NCH - 1, buf_a)

        pltpu.async_copy(outstage.at[pl.ds(slot * 6 * _KP, 6 * _KP)],
                         out_hbm.at[pl.ds(gb * 6 * _KP, 6 * _KP)], sem_o)
        return carry

    lax.fori_loop(0, _BPW, b_loop, 0)
    pltpu.make_async_copy(outstage.at[pl.ds(0, 6 * _KP)],
                          out_hbm.at[pl.ds((wid * _BPW + _BPW - 2) * 6 * _KP, 6 * _KP)], sem_o).wait()
    pltpu.make_async_copy(outstage.at[pl.ds(6 * _KP, 6 * _KP)],
                          out_hbm.at[pl.ds((wid * _BPW + _BPW - 1) * 6 * _KP, 6 * _KP)], sem_o).wait()


def _copy_body(a, b, c, d, oa, ob, oc, od):
    oa[...] = a[...]
    ob[...] = b[...]
    oc[...] = c[...]
    od[...] = d[...]


def _upd_body(y_ref, ml, mab, mo, mc, fl, fab, fo, fc,
              cl, cab, co, cc, ol, oab, oo, oc):
    del y_ref, cl, cab, co, cc
    for m, f, o in ((ml, fl, ol), (mab, fab, oab), (mo, fo, oo), (mc, fc, oc)):
        pos = m[...] * _MOM + f[...] * (1.0 - _MOM)
        norm = jnp.sqrt(jnp.sum(pos * pos, axis=-1, keepdims=True))
        o[...] = pos / norm


def kernel(l, ab, ori, comp, y, idx, memory_l, memory_ab, memory_ori, memory_comp):
    y = y.astype(jnp.int32)
    idxp = jnp.concatenate(
        [y[:, None], idx[:, 1:].astype(jnp.int32),
         jnp.zeros((_B, _KP - _K1), jnp.int32)], axis=1)
    featcat = jnp.concatenate([l, ab, ori, comp], axis=1)

    sc_fn = pl.kernel(
        _sc_body,
        out_type=jax.ShapeDtypeStruct((_B * 6 * _KP,), jnp.float32),
        mesh=plsc.VectorSubcoreMesh(core_axis_name="c", subcore_axis_name="s"),
        compiler_params=pltpu.CompilerParams(needs_layout_passes=False),
        scratch_types=[
            pltpu.VMEM((_BPW * _KP,), jnp.int32),
            pltpu.VMEM((_BPW * 4 * _D,), jnp.float32),
            pltpu.VMEM((_S, _D), jnp.float32),
            pltpu.VMEM((_S, _D), jnp.float32),
            pltpu.VMEM((_S, _D), jnp.float32),
            pltpu.VMEM((_S, _D), jnp.float32),
            pltpu.VMEM((_S, _D), jnp.float32),
            pltpu.VMEM((_S, _D), jnp.float32),
            pltpu.VMEM((_S, _D), jnp.float32),
            pltpu.VMEM((_S, _D), jnp.float32),
            pltpu.VMEM((2 * 6 * _KP,), jnp.float32),
        ] + [pltpu.SemaphoreType.DMA] * 9,
    )
    sc_out = sc_fn(featcat.reshape(-1), idxp.reshape(-1),
                   memory_l, memory_ab, memory_ori, memory_comp)

    rows_blk = 1000
    bank_sds = jax.ShapeDtypeStruct((_OUT, _D), jnp.float32)
    blk = pl.BlockSpec((rows_blk, _D), lambda i: (i, 0))
    copies = pl.pallas_call(
        _copy_body,
        grid=(_OUT // rows_blk,),
        in_specs=[blk] * 4,
        out_specs=[blk] * 4,
        out_shape=[bank_sds] * 4,
    )(memory_l, memory_ab, memory_ori, memory_comp)

    banks3 = [m.reshape(_OUT, 1, _D)
              for m in (memory_l, memory_ab, memory_ori, memory_comp)]
    feats3 = [f.reshape(_B, 1, _D) for f in (l, ab, ori, comp)]
    copies3 = [c.reshape(_OUT, 1, _D) for c in copies]
    bank3_sds = jax.ShapeDtypeStruct((_OUT, 1, _D), jnp.float32)
    row_y = pl.BlockSpec((1, 1, _D), lambda i, yref: (yref[i], 0, 0))
    row_i = pl.BlockSpec((1, 1, _D), lambda i, yref: (i, 0, 0))
    anyspec = pl.BlockSpec(memory_space=pl.ANY)
    grid_spec = pltpu.PrefetchScalarGridSpec(
        num_scalar_prefetch=1,
        grid=(_B,),
        in_specs=[row_y] * 4 + [row_i] * 4 + [anyspec] * 4,
        out_specs=[row_y] * 4,
    )
    new_banks = pl.pallas_call(
        _upd_body,
        grid_spec=grid_spec,
        out_shape=[bank3_sds] * 4,
        input_output_aliases={9: 0, 10: 1, 11: 2, 12: 3},
    )(y, *banks3, *feats3, *copies3)
    new_banks = [nb.reshape(_OUT, _D) for nb in new_banks]

    outs = sc_out.reshape(_B, 6, _KP)[:, :, :_K1]
    return (outs[:, 0][..., None], outs[:, 1][..., None], outs[:, 2][..., None],
            outs[:, 3][..., None], outs[:, 4][..., None], outs[:, 5][..., None],
            new_banks[0], new_banks[1], new_banks[2], new_banks[3])


# speedup vs baseline: 4.0851x; 2.5078x over previous
import jax
import jax.numpy as jnp
from jax import lax
from jax.experimental import pallas as pl
from jax.experimental.pallas import tpu as pltpu
from jax.experimental.pallas import tpu_sc as plsc

_B = 1024
_D = 128
_OUT = 100000
_K1 = 513
_T = 0.07
_MOM = 0.5

_NC, _NS, _L = 2, 16, 16
_NW = _NC * _NS
_BPW = _B // _NW
_S = 48
_KP = 528
_NCH = _KP // _S
_NG = _S // _L
_NDB = _D // _L
_FR = 4 * _NDB * _L * _L


def _sc_body(fr_hbm, idxp_hbm, ml_hbm, mab_hbm, mo_hbm, mc_hbm,
             out_hbm,
             idx_all, frow,
             wa_l, wa_ab, wa_o, wa_c, wb_l, wb_ab, wb_o, wb_c,
             outstage, sa0, sa1, sa2, sa3, sb0, sb1, sb2, sb3, sem_o):
    wid = lax.axis_index("s") * _NC + lax.axis_index("c")
    iota = lax.broadcasted_iota(jnp.int32, (_L,), 0)
    banks = (ml_hbm, mab_hbm, mo_hbm, mc_hbm)
    buf_a = (wa_l, wa_ab, wa_o, wa_c)
    buf_b = (wb_l, wb_ab, wb_o, wb_c)
    sems_a = (sa0, sa1, sa2, sa3)
    sems_b = (sb0, sb1, sb2, sb3)

    pltpu.sync_copy(idxp_hbm.at[pl.ds(wid * _BPW * _KP, _BPW * _KP)], idx_all)

    def issue(bi, c, bufs, sems):
        isl = idx_all.at[pl.ds(bi * _KP + c * _S, _S)]
        for m, w, sem in zip(banks, bufs, sems):
            pltpu.async_copy(m.at[isl], w, sem)

    def drain(bi, c, bufs, sems):
        isl = idx_all.at[pl.ds(bi * _KP + c * _S, _S)]
        for m, w, sem in zip(banks, bufs, sems):
            pltpu.make_async_copy(m.at[isl], w, sem).wait()

    def compute(bi, c, bufs):
        wl, wab, wo, wc = bufs
        slot = lax.rem(bi, 2)

        def g_loop(g, _):
            riota = iota + g * _L

            def d_loop(dd, accs):
                a0, a1, a2, a3, a4, a5 = accs
                d0 = dd * _L
                for u in range(_L):
                    dvec = d0 + lax.rem(iota + u, _L)
                    cl = plsc.load_gather(wl, [riota, dvec])
                    cab = plsc.load_gather(wab, [riota, dvec])
                    co = plsc.load_gather(wo, [riota, dvec])
                    cc = plsc.load_gather(wc, [riota, dvec])
                    vfl = frow[pl.ds((dd * _L + u) * _L, _L)]
                    vfab = frow[pl.ds((_NDB * _L + dd * _L + u) * _L, _L)]
                    vfo = frow[pl.ds((2 * _NDB * _L + dd * _L + u) * _L, _L)]
                    vfc = frow[pl.ds((3 * _NDB * _L + dd * _L + u) * _L, _L)]
                    a0 = a0 + co * vfl
                    a1 = a1 + cl * vfab
                    a2 = a2 + cab * vfo
                    a3 = a3 + cc * vfab
                    a4 = a4 + cc * vfl
                    a5 = a5 + co * vfc
                return (a0, a1, a2, a3, a4, a5)

            z = jnp.zeros((_L,), jnp.float32)
            accs = lax.fori_loop(0, _NDB, d_loop, (z, z, z, z, z, z))
            for t in range(6):
                outstage[pl.ds(slot * 6 * _KP + t * _KP + c * _S + g * _L,
                               _L)] = accs[t]
            return _

        lax.fori_loop(0, _NG, g_loop, 0)

    def b_loop(bi, carry):
        gb = wid * _BPW + bi
        slot = lax.rem(bi, 2)

        @pl.when(bi >= 2)
        def _wait_prev():
            pltpu.make_async_copy(outstage.at[pl.ds(slot * 6 * _KP, 6 * _KP)],
                                  out_hbm.at[pl.ds((gb - 2) * 6 * _KP, 6 * _KP)], sem_o).wait()

        issue(bi, 0, buf_a, sems_a)
        pltpu.sync_copy(fr_hbm.at[pl.ds(gb * _FR, _FR)], frow)

        def t_loop(t, _):
            c0 = 2 * t
            issue(bi, c0 + 1, buf_b, sems_b)
            drain(bi, c0, buf_a, sems_a)
            compute(bi, c0, buf_a)
            issue(bi, c0 + 2, buf_a, sems_a)
            drain(bi, c0 + 1, buf_b, sems_b)
            compute(bi, c0 + 1, buf_b)
            return _

        lax.fori_loop(0, (_NCH - 1) // 2, t_loop, 0)
        drain(bi, _NCH - 1, buf_a, sems_a)
        compute(bi, _NCH - 1, buf_a)

        pltpu.async_copy(outstage.at[pl.ds(slot * 6 * _KP, 6 * _KP)],
                         out_hbm.at[pl.ds(gb * 6 * _KP, 6 * _KP)], sem_o)
        return carry

    lax.fori_loop(0, _BPW, b_loop, 0)
    pltpu.make_async_copy(outstage.at[pl.ds(0, 6 * _KP)],
                          out_hbm.at[pl.ds((wid * _BPW + _BPW - 2) * 6 * _KP, 6 * _KP)], sem_o).wait()
    pltpu.make_async_copy(outstage.at[pl.ds(6 * _KP, 6 * _KP)],
                          out_hbm.at[pl.ds((wid * _BPW + _BPW - 1) * 6 * _KP, 6 * _KP)], sem_o).wait()


def _copy_body(a, b, c, d, oa, ob, oc, od):
    oa[...] = a[...]
    ob[...] = b[...]
    oc[...] = c[...]
    od[...] = d[...]


def _upd_body(y_ref, ml, mab, mo, mc, fl, fab, fo, fc,
              cl, cab, co, cc, ol, oab, oo, oc):
    del y_ref, cl, cab, co, cc
    for m, f, o in ((ml, fl, ol), (mab, fab, oab), (mo, fo, oo), (mc, fc, oc)):
        pos = m[...] * _MOM + f[...] * (1.0 - _MOM)
        norm = jnp.sqrt(jnp.sum(pos * pos, axis=-1, keepdims=True))
        o[...] = pos / norm


def kernel(l, ab, ori, comp, y, idx, memory_l, memory_ab, memory_ori, memory_comp):
    y = y.astype(jnp.int32)
    idxp = jnp.concatenate(
        [y[:, None], idx[:, 1:].astype(jnp.int32),
         jnp.zeros((_B, _KP - _K1), jnp.int32)], axis=1)
    feats = jnp.stack([l, ab, ori, comp], axis=1) * (1.0 / _T)
    rot = (jnp.arange(_L)[:, None] + jnp.arange(_L)[None, :]) % _L
    fr = feats.reshape(_B, 4, _NDB, _L)[:, :, :, rot]

    sc_fn = pl.kernel(
        _sc_body,
        out_type=jax.ShapeDtypeStruct((_B * 6 * _KP,), jnp.float32),
        mesh=plsc.VectorSubcoreMesh(core_axis_name="c", subcore_axis_name="s"),
        compiler_params=pltpu.CompilerParams(needs_layout_passes=False),
        scratch_types=[
            pltpu.VMEM((_BPW * _KP,), jnp.int32),
            pltpu.VMEM((_FR,), jnp.float32),
            pltpu.VMEM((_S, _D), jnp.float32),
            pltpu.VMEM((_S, _D), jnp.float32),
            pltpu.VMEM((_S, _D), jnp.float32),
            pltpu.VMEM((_S, _D), jnp.float32),
            pltpu.VMEM((_S, _D), jnp.float32),
            pltpu.VMEM((_S, _D), jnp.float32),
            pltpu.VMEM((_S, _D), jnp.float32),
            pltpu.VMEM((_S, _D), jnp.float32),
            pltpu.VMEM((2 * 6 * _KP,), jnp.float32),
        ] + [pltpu.SemaphoreType.DMA] * 9,
    )
    sc_out = sc_fn(fr.reshape(-1), idxp.reshape(-1),
                   memory_l, memory_ab, memory_ori, memory_comp)

    rows_blk = 1000
    bank_sds = jax.ShapeDtypeStruct((_OUT, _D), jnp.float32)
    blk = pl.BlockSpec((rows_blk, _D), lambda i: (i, 0))
    copies = pl.pallas_call(
        _copy_body,
        grid=(_OUT // rows_blk,),
        in_specs=[blk] * 4,
        out_specs=[blk] * 4,
        out_shape=[bank_sds] * 4,
    )(memory_l, memory_ab, memory_ori, memory_comp)

    banks3 = [m.reshape(_OUT, 1, _D)
              for m in (memory_l, memory_ab, memory_ori, memory_comp)]
    feats3 = [f.reshape(_B, 1, _D) for f in (l, ab, ori, comp)]
    copies3 = [c.reshape(_OUT, 1, _D) for c in copies]
    bank3_sds = jax.ShapeDtypeStruct((_OUT, 1, _D), jnp.float32)
    row_y = pl.BlockSpec((1, 1, _D), lambda i, yref: (yref[i], 0, 0))
    row_i = pl.BlockSpec((1, 1, _D), lambda i, yref: (i, 0, 0))
    anyspec = pl.BlockSpec(memory_space=pl.ANY)
    grid_spec = pltpu.PrefetchScalarGridSpec(
        num_scalar_prefetch=1,
        grid=(_B,),
        in_specs=[row_y] * 4 + [row_i] * 4 + [anyspec] * 4,
        out_specs=[row_y] * 4,
    )
    new_banks = pl.pallas_call(
        _upd_body,
        grid_spec=grid_spec,
        out_shape=[bank3_sds] * 4,
        input_output_aliases={9: 0, 10: 1, 11: 2, 12: 3},
    )(y, *banks3, *feats3, *copies3)
    new_banks = [nb.reshape(_OUT, _D) for nb in new_banks]

    outs = sc_out.reshape(_B, 6, _KP)[:, :, :_K1]
    return (outs[:, 0][..., None], outs[:, 1][..., None], outs[:, 2][..., None],
            outs[:, 3][..., None], outs[:, 4][..., None], outs[:, 5][..., None],
            new_banks[0], new_banks[1], new_banks[2], new_banks[3])


# speedup vs baseline: 5.4993x; 1.3462x over previous
import jax
import jax.numpy as jnp
from jax import lax
from jax.experimental import pallas as pl
from jax.experimental.pallas import tpu as pltpu
from jax.experimental.pallas import tpu_sc as plsc

_B = 1024
_D = 128
_OUT = 100000
_K1 = 513
_T = 0.07
_MOM = 0.5

_NC, _NS, _L = 2, 16, 16
_NW = _NC * _NS
_BPW = _B // _NW
_S = 48
_KP = 528
_NCH = _KP // _S
_NG = _S // _L
_NDB = _D // _L
_FR = 4 * _NDB * _L * _L


def _sc_body(fr_hbm, idxp_hbm, ml_hbm, mab_hbm, mo_hbm, mc_hbm,
             out_hbm,
             idx_all, frow,
             wa_l, wa_ab, wa_o, wa_c, wb_l, wb_ab, wb_o, wb_c,
             outstage, sa0, sa1, sa2, sa3, sb0, sb1, sb2, sb3, sem_o):
    wid = lax.axis_index("s") * _NC + lax.axis_index("c")
    iota = lax.broadcasted_iota(jnp.int32, (_L,), 0)
    banks = (ml_hbm, mab_hbm, mo_hbm, mc_hbm)
    buf_a = (wa_l, wa_ab, wa_o, wa_c)
    buf_b = (wb_l, wb_ab, wb_o, wb_c)
    sems_a = (sa0, sa1, sa2, sa3)
    sems_b = (sb0, sb1, sb2, sb3)

    pltpu.sync_copy(idxp_hbm.at[pl.ds(wid * _BPW * _KP, _BPW * _KP)], idx_all)

    def issue(bi, c, bufs, sems):
        isl = idx_all.at[pl.ds(bi * _KP + c * _S, _S)]
        for m, w, sem in zip(banks, bufs, sems):
            pltpu.async_copy(m.at[isl], w, sem)

    def drain(bi, c, bufs, sems):
        isl = idx_all.at[pl.ds(bi * _KP + c * _S, _S)]
        for m, w, sem in zip(banks, bufs, sems):
            pltpu.make_async_copy(m.at[isl], w, sem).wait()

    def compute(bi, c, bufs):
        wl, wab, wo, wc = bufs
        slot = lax.rem(bi, 2)

        def d_loop(dd, accs):
            accs = list(accs)
            d0 = dd * _L
            for u in range(_L):
                dvec = d0 + lax.rem(iota + u, _L)
                vfl = frow[pl.ds((dd * _L + u) * _L, _L)]
                vfab = frow[pl.ds((_NDB * _L + dd * _L + u) * _L, _L)]
                vfo = frow[pl.ds((2 * _NDB * _L + dd * _L + u) * _L, _L)]
                vfc = frow[pl.ds((3 * _NDB * _L + dd * _L + u) * _L, _L)]
                for g in range(_NG):
                    riota = iota + g * _L
                    cl = plsc.load_gather(wl, [riota, dvec])
                    cab = plsc.load_gather(wab, [riota, dvec])
                    co = plsc.load_gather(wo, [riota, dvec])
                    cc = plsc.load_gather(wc, [riota, dvec])
                    a0, a1, a2, a3, a4, a5 = accs[g * 6:(g + 1) * 6]
                    accs[g * 6:(g + 1) * 6] = [
                        a0 + co * vfl,
                        a1 + cl * vfab,
                        a2 + cab * vfo,
                        a3 + cc * vfab,
                        a4 + cc * vfl,
                        a5 + co * vfc,
                    ]
            return tuple(accs)

        z = jnp.zeros((_L,), jnp.float32)
        accs = lax.fori_loop(0, _NDB, d_loop, (z,) * (6 * _NG))
        for g in range(_NG):
            for t in range(6):
                outstage[pl.ds(slot * 6 * _KP + t * _KP + c * _S + g * _L,
                               _L)] = accs[g * 6 + t]

    def b_loop(bi, carry):
        gb = wid * _BPW + bi
        slot = lax.rem(bi, 2)

        @pl.when(bi >= 2)
        def _wait_prev():
            pltpu.make_async_copy(outstage.at[pl.ds(slot * 6 * _KP, 6 * _KP)],
                                  out_hbm.at[pl.ds((gb - 2) * 6 * _KP, 6 * _KP)], sem_o).wait()

        issue(bi, 0, buf_a, sems_a)
        pltpu.sync_copy(fr_hbm.at[pl.ds(gb * _FR, _FR)], frow)

        def t_loop(t, _):
            c0 = 2 * t
            issue(bi, c0 + 1, buf_b, sems_b)
            drain(bi, c0, buf_a, sems_a)
            compute(bi, c0, buf_a)
            issue(bi, c0 + 2, buf_a, sems_a)
            drain(bi, c0 + 1, buf_b, sems_b)
            compute(bi, c0 + 1, buf_b)
            return _

        lax.fori_loop(0, (_NCH - 1) // 2, t_loop, 0)
        drain(bi, _NCH - 1, buf_a, sems_a)
        compute(bi, _NCH - 1, buf_a)

        pltpu.async_copy(outstage.at[pl.ds(slot * 6 * _KP, 6 * _KP)],
                         out_hbm.at[pl.ds(gb * 6 * _KP, 6 * _KP)], sem_o)
        return carry

    lax.fori_loop(0, _BPW, b_loop, 0)
    pltpu.make_async_copy(outstage.at[pl.ds(0, 6 * _KP)],
                          out_hbm.at[pl.ds((wid * _BPW + _BPW - 2) * 6 * _KP, 6 * _KP)], sem_o).wait()
    pltpu.make_async_copy(outstage.at[pl.ds(6 * _KP, 6 * _KP)],
                          out_hbm.at[pl.ds((wid * _BPW + _BPW - 1) * 6 * _KP, 6 * _KP)], sem_o).wait()


def _copy_body(a, b, c, d, oa, ob, oc, od):
    oa[...] = a[...]
    ob[...] = b[...]
    oc[...] = c[...]
    od[...] = d[...]


def _upd_body(y_ref, ml, mab, mo, mc, fl, fab, fo, fc,
              cl, cab, co, cc, ol, oab, oo, oc):
    del y_ref, cl, cab, co, cc
    for m, f, o in ((ml, fl, ol), (mab, fab, oab), (mo, fo, oo), (mc, fc, oc)):
        pos = m[...] * _MOM + f[...] * (1.0 - _MOM)
        norm = jnp.sqrt(jnp.sum(pos * pos, axis=-1, keepdims=True))
        o[...] = pos / norm


def kernel(l, ab, ori, comp, y, idx, memory_l, memory_ab, memory_ori, memory_comp):
    y = y.astype(jnp.int32)
    idxp = jnp.concatenate(
        [y[:, None], idx[:, 1:].astype(jnp.int32),
         jnp.zeros((_B, _KP - _K1), jnp.int32)], axis=1)
    feats = jnp.stack([l, ab, ori, comp], axis=1) * (1.0 / _T)
    rot = (jnp.arange(_L)[:, None] + jnp.arange(_L)[None, :]) % _L
    fr = feats.reshape(_B, 4, _NDB, _L)[:, :, :, rot]

    sc_fn = pl.kernel(
        _sc_body,
        out_type=jax.ShapeDtypeStruct((_B * 6 * _KP,), jnp.float32),
        mesh=plsc.VectorSubcoreMesh(core_axis_name="c", subcore_axis_name="s"),
        compiler_params=pltpu.CompilerParams(needs_layout_passes=False),
        scratch_types=[
            pltpu.VMEM((_BPW * _KP,), jnp.int32),
            pltpu.VMEM((_FR,), jnp.float32),
            pltpu.VMEM((_S, _D), jnp.float32),
            pltpu.VMEM((_S, _D), jnp.float32),
            pltpu.VMEM((_S, _D), jnp.float32),
            pltpu.VMEM((_S, _D), jnp.float32),
            pltpu.VMEM((_S, _D), jnp.float32),
            pltpu.VMEM((_S, _D), jnp.float32),
            pltpu.VMEM((_S, _D), jnp.float32),
            pltpu.VMEM((_S, _D), jnp.float32),
            pltpu.VMEM((2 * 6 * _KP,), jnp.float32),
        ] + [pltpu.SemaphoreType.DMA] * 9,
    )
    sc_out = sc_fn(fr.reshape(-1), idxp.reshape(-1),
                   memory_l, memory_ab, memory_ori, memory_comp)

    rows_blk = 1000
    bank_sds = jax.ShapeDtypeStruct((_OUT, _D), jnp.float32)
    blk = pl.BlockSpec((rows_blk, _D), lambda i: (i, 0))
    copies = pl.pallas_call(
        _copy_body,
        grid=(_OUT // rows_blk,),
        in_specs=[blk] * 4,
        out_specs=[blk] * 4,
        out_shape=[bank_sds] * 4,
    )(memory_l, memory_ab, memory_ori, memory_comp)

    banks3 = [m.reshape(_OUT, 1, _D)
              for m in (memory_l, memory_ab, memory_ori, memory_comp)]
    feats3 = [f.reshape(_B, 1, _D) for f in (l, ab, ori, comp)]
    copies3 = [c.reshape(_OUT, 1, _D) for c in copies]
    bank3_sds = jax.ShapeDtypeStruct((_OUT, 1, _D), jnp.float32)
    row_y = pl.BlockSpec((1, 1, _D), lambda i, yref: (yref[i], 0, 0))
    row_i = pl.BlockSpec((1, 1, _D), lambda i, yref: (i, 0, 0))
    anyspec = pl.BlockSpec(memory_space=pl.ANY)
    grid_spec = pltpu.PrefetchScalarGridSpec(
        num_scalar_prefetch=1,
        grid=(_B,),
        in_specs=[row_y] * 4 + [row_i] * 4 + [anyspec] * 4,
        out_specs=[row_y] * 4,
    )
    new_banks = pl.pallas_call(
        _upd_body,
        grid_spec=grid_spec,
        out_shape=[bank3_sds] * 4,
        input_output_aliases={9: 0, 10: 1, 11: 2, 12: 3},
    )(y, *banks3, *feats3, *copies3)
    new_banks = [nb.reshape(_OUT, _D) for nb in new_banks]

    outs = sc_out.reshape(_B, 6, _KP)[:, :, :_K1]
    return (outs[:, 0][..., None], outs[:, 1][..., None], outs[:, 2][..., None],
            outs[:, 3][..., None], outs[:, 4][..., None], outs[:, 5][..., None],
            new_banks[0], new_banks[1], new_banks[2], new_banks[3])


# speedup vs baseline: 6.4063x; 1.1649x over previous
import jax
import jax.numpy as jnp
from jax import lax
from jax.experimental import pallas as pl
from jax.experimental.pallas import tpu as pltpu
from jax.experimental.pallas import tpu_sc as plsc

_B = 1024
_D = 128
_OUT = 100000
_K1 = 513
_T = 0.07
_MOM = 0.5

_NC, _NS, _L = 2, 16, 16
_NW = _NC * _NS
_BPW = _B // _NW
_S = 48
_KP = 528
_NCH = _KP // _S
_NG = _S // _L
_NDB = _D // _L
_FR = 4 * _D


def _sc_body(fr_hbm, idxp_hbm, ml_hbm, mab_hbm, mo_hbm, mc_hbm,
             out_hbm,
             idx_all, frow,
             wa_l, wa_ab, wa_o, wa_c, wb_l, wb_ab, wb_o, wb_c,
             outstage, sa0, sa1, sa2, sa3, sb0, sb1, sb2, sb3, sem_o):
    wid = lax.axis_index("s") * _NC + lax.axis_index("c")
    iota = lax.broadcasted_iota(jnp.int32, (_L,), 0)
    banks = (ml_hbm, mab_hbm, mo_hbm, mc_hbm)
    buf_a = (wa_l, wa_ab, wa_o, wa_c)
    buf_b = (wb_l, wb_ab, wb_o, wb_c)
    sems_a = (sa0, sa1, sa2, sa3)
    sems_b = (sb0, sb1, sb2, sb3)

    pltpu.sync_copy(idxp_hbm.at[pl.ds(wid * _BPW * _KP, _BPW * _KP)], idx_all)

    def issue(bi, c, bufs, sems):
        isl = idx_all.at[pl.ds(bi * _KP + c * _S, _S)]
        for m, w, sem in zip(banks, bufs, sems):
            pltpu.async_copy(m.at[isl], w, sem)

    def drain(bi, c, bufs, sems):
        isl = idx_all.at[pl.ds(bi * _KP + c * _S, _S)]
        for m, w, sem in zip(banks, bufs, sems):
            pltpu.make_async_copy(m.at[isl], w, sem).wait()

    def compute(bi, c, bufs):
        wl, wab, wo, wc = bufs
        slot = lax.rem(bi, 2)

        def d_loop(dd, accs):
            accs = list(accs)
            d0 = dd * _L
            for u in range(_L):
                dvec = d0 + lax.rem(iota + u, _L)
                vfl = plsc.load_gather(frow, [dvec])
                vfab = plsc.load_gather(frow, [dvec + _D])
                vfo = plsc.load_gather(frow, [dvec + 2 * _D])
                vfc = plsc.load_gather(frow, [dvec + 3 * _D])
                for g in range(_NG):
                    riota = iota + g * _L
                    cl = plsc.load_gather(wl, [riota, dvec])
                    cab = plsc.load_gather(wab, [riota, dvec])
                    co = plsc.load_gather(wo, [riota, dvec])
                    cc = plsc.load_gather(wc, [riota, dvec])
                    a0, a1, a2, a3, a4, a5 = accs[g * 6:(g + 1) * 6]
                    accs[g * 6:(g + 1) * 6] = [
                        a0 + co * vfl,
                        a1 + cl * vfab,
                        a2 + cab * vfo,
                        a3 + cc * vfab,
                        a4 + cc * vfl,
                        a5 + co * vfc,
                    ]
            return tuple(accs)

        z = jnp.zeros((_L,), jnp.float32)
        accs = lax.fori_loop(0, _NDB, d_loop, (z,) * (6 * _NG))
        for g in range(_NG):
            for t in range(6):
                outstage[pl.ds(slot * 6 * _KP + t * _KP + c * _S + g * _L,
                               _L)] = accs[g * 6 + t]

    def b_loop(bi, carry):
        gb = wid * _BPW + bi
        slot = lax.rem(bi, 2)

        @pl.when(bi >= 2)
        def _wait_prev():
            pltpu.make_async_copy(outstage.at[pl.ds(slot * 6 * _KP, 6 * _KP)],
                                  out_hbm.at[pl.ds((gb - 2) * 6 * _KP, 6 * _KP)], sem_o).wait()

        issue(bi, 0, buf_a, sems_a)
        pltpu.sync_copy(fr_hbm.at[pl.ds(gb * _FR, _FR)], frow)

        def t_loop(t, _):
            c0 = 2 * t
            issue(bi, c0 + 1, buf_b, sems_b)
            drain(bi, c0, buf_a, sems_a)
            compute(bi, c0, buf_a)
            issue(bi, c0 + 2, buf_a, sems_a)
            drain(bi, c0 + 1, buf_b, sems_b)
            compute(bi, c0 + 1, buf_b)
            return _

        lax.fori_loop(0, (_NCH - 1) // 2, t_loop, 0)
        drain(bi, _NCH - 1, buf_a, sems_a)
        compute(bi, _NCH - 1, buf_a)

        pltpu.async_copy(outstage.at[pl.ds(slot * 6 * _KP, 6 * _KP)],
                         out_hbm.at[pl.ds(gb * 6 * _KP, 6 * _KP)], sem_o)
        return carry

    lax.fori_loop(0, _BPW, b_loop, 0)
    pltpu.make_async_copy(outstage.at[pl.ds(0, 6 * _KP)],
                          out_hbm.at[pl.ds((wid * _BPW + _BPW - 2) * 6 * _KP, 6 * _KP)], sem_o).wait()
    pltpu.make_async_copy(outstage.at[pl.ds(6 * _KP, 6 * _KP)],
                          out_hbm.at[pl.ds((wid * _BPW + _BPW - 1) * 6 * _KP, 6 * _KP)], sem_o).wait()


def _copy_body(a, b, c, d, oa, ob, oc, od):
    oa[...] = a[...]
    ob[...] = b[...]
    oc[...] = c[...]
    od[...] = d[...]


def _upd_body(y_ref, ml, mab, mo, mc, fl, fab, fo, fc,
              cl, cab, co, cc, ol, oab, oo, oc):
    del y_ref, cl, cab, co, cc
    for m, f, o in ((ml, fl, ol), (mab, fab, oab), (mo, fo, oo), (mc, fc, oc)):
        pos = m[...] * _MOM + f[...] * (1.0 - _MOM)
        norm = jnp.sqrt(jnp.sum(pos * pos, axis=-1, keepdims=True))
        o[...] = pos / norm


def kernel(l, ab, ori, comp, y, idx, memory_l, memory_ab, memory_ori, memory_comp):
    y = y.astype(jnp.int32)
    idxp = jnp.concatenate(
        [y[:, None], idx[:, 1:].astype(jnp.int32),
         jnp.zeros((_B, _KP - _K1), jnp.int32)], axis=1)
    fr = jnp.stack([l, ab, ori, comp], axis=1) * (1.0 / _T)

    sc_fn = pl.kernel(
        _sc_body,
        out_type=jax.ShapeDtypeStruct((_B * 6 * _KP,), jnp.float32),
        mesh=plsc.VectorSubcoreMesh(core_axis_name="c", subcore_axis_name="s"),
        compiler_params=pltpu.CompilerParams(needs_layout_passes=False),
        scratch_types=[
            pltpu.VMEM((_BPW * _KP,), jnp.int32),
            pltpu.VMEM((_FR,), jnp.float32),
            pltpu.VMEM((_S, _D), jnp.float32),
            pltpu.VMEM((_S, _D), jnp.float32),
            pltpu.VMEM((_S, _D), jnp.float32),
            pltpu.VMEM((_S, _D), jnp.float32),
            pltpu.VMEM((_S, _D), jnp.float32),
            pltpu.VMEM((_S, _D), jnp.float32),
            pltpu.VMEM((_S, _D), jnp.float32),
            pltpu.VMEM((_S, _D), jnp.float32),
            pltpu.VMEM((2 * 6 * _KP,), jnp.float32),
        ] + [pltpu.SemaphoreType.DMA] * 9,
    )
    sc_out = sc_fn(fr.reshape(-1), idxp.reshape(-1),
                   memory_l, memory_ab, memory_ori, memory_comp)

    rows_blk = 1000
    bank_sds = jax.ShapeDtypeStruct((_OUT, _D), jnp.float32)
    blk = pl.BlockSpec((rows_blk, _D), lambda i: (i, 0))
    copies = pl.pallas_call(
        _copy_body,
        grid=(_OUT // rows_blk,),
        in_specs=[blk] * 4,
        out_specs=[blk] * 4,
        out_shape=[bank_sds] * 4,
    )(memory_l, memory_ab, memory_ori, memory_comp)

    banks3 = [m.reshape(_OUT, 1, _D)
              for m in (memory_l, memory_ab, memory_ori, memory_comp)]
    feats3 = [f.reshape(_B, 1, _D) for f in (l, ab, ori, comp)]
    copies3 = [c.reshape(_OUT, 1, _D) for c in copies]
    bank3_sds = jax.ShapeDtypeStruct((_OUT, 1, _D), jnp.float32)
    row_y = pl.BlockSpec((1, 1, _D), lambda i, yref: (yref[i], 0, 0))
    row_i = pl.BlockSpec((1, 1, _D), lambda i, yref: (i, 0, 0))
    anyspec = pl.BlockSpec(memory_space=pl.ANY)
    grid_spec = pltpu.PrefetchScalarGridSpec(
        num_scalar_prefetch=1,
        grid=(_B,),
        in_specs=[row_y] * 4 + [row_i] * 4 + [anyspec] * 4,
        out_specs=[row_y] * 4,
    )
    new_banks = pl.pallas_call(
        _upd_body,
        grid_spec=grid_spec,
        out_shape=[bank3_sds] * 4,
        input_output_aliases={9: 0, 10: 1, 11: 2, 12: 3},
    )(y, *banks3, *feats3, *copies3)
    new_banks = [nb.reshape(_OUT, _D) for nb in new_banks]

    outs = sc_out.reshape(_B, 6, _KP)[:, :, :_K1]
    return (outs[:, 0][..., None], outs[:, 1][..., None], outs[:, 2][..., None],
            outs[:, 3][..., None], outs[:, 4][..., None], outs[:, 5][..., None],
            new_banks[0], new_banks[1], new_banks[2], new_banks[3])
